# baseline (speedup 1.0000x reference)
import jax
import jax.numpy as jnp
from jax import lax
from jax.experimental import pallas as pl
from jax.experimental.pallas import tpu as pltpu
from jax.experimental.pallas import tpu_sc as plsc

N_NODES = 10000
D = 128
DP = D // 2
E = 320000

_INFO = plsc.get_sparse_core_info()
NC, NS, L = _INFO.num_cores, _INFO.num_subcores, _INFO.num_lanes
NW = NC * NS
EPW = E // NW
C = 80
NCHUNK = EPW // C


def _edge_scores_body(xs_hbm, xm_hbm, xr_hbm, a_hbm, b_hbm, c_hbm,
                      o1_hbm, o2_hbm,
                      ia, ib, ic, rows, o1f, o2f, sem0, sem1):
    wid = lax.axis_index("s") * NC + lax.axis_index("c")
    base_w = wid * EPW
    pltpu.sync_copy(a_hbm.at[pl.ds(base_w, EPW)], ia)
    pltpu.sync_copy(b_hbm.at[pl.ds(base_w, EPW)], ib)
    pltpu.sync_copy(c_hbm.at[pl.ds(base_w, EPW)], ic)
    sems = (sem0, sem1)
    tables = (xs_hbm, xm_hbm, xr_hbm)
    idxs = (ia, ib, ic)
    lane = lax.iota(jnp.int32, L)

    def copies(c, p):
        return [pltpu.make_async_copy(
                    tables[t].at[idxs[t].at[pl.ds(c * C, C)]],
                    rows.at[p, t], sems[p])
                for t in range(3)]

    def start(c, p):
        for cp in copies(c, p):
            cp.start()

    def compute(c, p):
        for cp in copies(c, p):
            cp.wait()
        rs, rm, rr = rows.at[p, 0], rows.at[p, 1], rows.at[p, 2]
        obase = c * C

        @plsc.parallel_loop(0, C // L, unroll=2)
        def group_body(g):
            row = g * L + lane
            acc1 = jnp.zeros((L,), jnp.float32)
            acc2 = jnp.zeros((L,), jnp.float32)
            for d in range(DP):
                col = lane ^ d
                mp = plsc.load_gather(rm, [row, col])
                sp = plsc.load_gather(rs, [row, col])
                rp = plsc.load_gather(rr, [row, col])
                ma, mb = plsc.unpack(plsc.bitcast(mp, jnp.bfloat16),
                                     format=plsc.PackFormat.INTERLEAVED)
                sa, sb = plsc.unpack(plsc.bitcast(sp, jnp.bfloat16),
                                     format=plsc.PackFormat.INTERLEAVED)
                ra, rb = plsc.unpack(plsc.bitcast(rp, jnp.bfloat16),
                                     format=plsc.PackFormat.INTERLEAVED)
                acc1 = acc1 + sa * ma + sb * mb
                acc2 = acc2 + ra * ma + rb * mb
            o1f[pl.ds(obase + g * L, L)] = acc1
            o2f[pl.ds(obase + g * L, L)] = acc2

    start(0, 0)

    def body2(k2, carry):
        c0 = 2 * k2
        start(c0 + 1, 1)
        compute(c0, 0)
        start(c0 + 2, 0)
        compute(c0 + 1, 1)
        return carry

    lax.fori_loop(0, (NCHUNK - 1) // 2, body2, 0)
    compute(NCHUNK - 1, 0)
    pltpu.sync_copy(o1f, o1_hbm.at[pl.ds(base_w, EPW)])
    pltpu.sync_copy(o2f, o2_hbm.at[pl.ds(base_w, EPW)])


def _pack_table(x):
    xb = x.astype(jnp.bfloat16).reshape(N_NODES, DP, 2)
    return jax.lax.bitcast_convert_type(xb, jnp.int32)


@jax.jit
def _edge_scores(xs, xm, xr, a, b, c):
    f32 = jnp.float32
    run = pl.kernel(
        _edge_scores_body,
        out_type=(jax.ShapeDtypeStruct((E,), f32),
                  jax.ShapeDtypeStruct((E,), f32)),
        mesh=plsc.VectorSubcoreMesh(core_axis_name="c", subcore_axis_name="s"),
        compiler_params=pltpu.CompilerParams(needs_layout_passes=False,
                                             use_tc_tiling_on_sc=False),
        scratch_types=[
            pltpu.VMEM((EPW,), jnp.int32),
            pltpu.VMEM((EPW,), jnp.int32),
            pltpu.VMEM((EPW,), jnp.int32),
            pltpu.VMEM((2, 3, C, DP), jnp.int32),
            pltpu.VMEM((EPW,), f32),
            pltpu.VMEM((EPW,), f32),
            pltpu.SemaphoreType.DMA,
            pltpu.SemaphoreType.DMA,
        ],
    )
    return run(xs, xm, xr, a, b, c)


def kernel(x_srna, x_mrna, x_rbp, edge_label_index, edge_label_index_rbp):
    a = edge_label_index[0].astype(jnp.int32)
    b = edge_label_index[1].astype(jnp.int32)
    c = edge_label_index_rbp[0].astype(jnp.int32)
    return _edge_scores(_pack_table(x_srna), _pack_table(x_mrna),
                        _pack_table(x_rbp), a, b, c)


# speedup vs baseline: 1.3317x; 1.3317x over previous
import jax
import jax.numpy as jnp
from jax import lax
from jax.experimental import pallas as pl
from jax.experimental.pallas import tpu as pltpu
from jax.experimental.pallas import tpu_sc as plsc

N_NODES = 10000
D = 128
DP = D // 2
E = 320000

_INFO = plsc.get_sparse_core_info()
NC, NS, L = _INFO.num_cores, _INFO.num_subcores, _INFO.num_lanes
NW = NC * NS
EPW = E // NW
C = 80
NCHUNK = EPW // C


def _edge_scores_body(xs_hbm, xm_hbm, xr_hbm, a_hbm, b_hbm, c_hbm,
                      o1_hbm, o2_hbm,
                      ia, ib, ic, rows, o1f, o2f, sem0, sem1):
    wid = lax.axis_index("s") * NC + lax.axis_index("c")
    base_w = wid * EPW
    pltpu.sync_copy(a_hbm.at[pl.ds(base_w, EPW)], ia)
    pltpu.sync_copy(b_hbm.at[pl.ds(base_w, EPW)], ib)
    pltpu.sync_copy(c_hbm.at[pl.ds(base_w, EPW)], ic)
    sems = (sem0, sem1)
    tables = (xs_hbm, xm_hbm, xr_hbm)
    idxs = (ia, ib, ic)
    lane = lax.iota(jnp.int32, L)

    def copies(c, p):
        return [pltpu.make_async_copy(
                    tables[t].at[idxs[t].at[pl.ds(c * C, C)]],
                    rows.at[p, t], sems[p])
                for t in range(3)]

    def start(c, p):
        for cp in copies(c, p):
            cp.start()

    def compute(c, p):
        for cp in copies(c, p):
            cp.wait()
        rs, rm, rr = rows.at[p, 0], rows.at[p, 1], rows.at[p, 2]
        obase = c * C

        def group_body(g, carry):
            row = g * L + lane
            acc1 = jnp.zeros((L,), jnp.float32)
            acc2 = jnp.zeros((L,), jnp.float32)
            for d in range(DP):
                col = lane ^ d
                mp = plsc.load_gather(rm, [row, col])
                sp = plsc.load_gather(rs, [row, col])
                rp = plsc.load_gather(rr, [row, col])
                ma, mb = plsc.unpack(plsc.bitcast(mp, jnp.bfloat16),
                                     format=plsc.PackFormat.INTERLEAVED)
                sa, sb = plsc.unpack(plsc.bitcast(sp, jnp.bfloat16),
                                     format=plsc.PackFormat.INTERLEAVED)
                ra, rb = plsc.unpack(plsc.bitcast(rp, jnp.bfloat16),
                                     format=plsc.PackFormat.INTERLEAVED)
                acc1 = acc1 + sa * ma + sb * mb
                acc2 = acc2 + ra * ma + rb * mb
            o1f[pl.ds(obase + g * L, L)] = acc1
            o2f[pl.ds(obase + g * L, L)] = acc2
            return carry

        lax.fori_loop(0, C // L, group_body, 0)

    start(0, 0)

    def body2(k2, carry):
        c0 = 2 * k2
        start(c0 + 1, 1)
        compute(c0, 0)
        start(c0 + 2, 0)
        compute(c0 + 1, 1)
        return carry

    lax.fori_loop(0, (NCHUNK - 1) // 2, body2, 0)
    compute(NCHUNK - 1, 0)
    pltpu.sync_copy(o1f, o1_hbm.at[pl.ds(base_w, EPW)])
    pltpu.sync_copy(o2f, o2_hbm.at[pl.ds(base_w, EPW)])


def _pack_table(x):
    xb = x.astype(jnp.bfloat16).reshape(N_NODES, DP, 2)
    return jax.lax.bitcast_convert_type(xb, jnp.int32)


@jax.jit
def _edge_scores(xs, xm, xr, a, b, c):
    f32 = jnp.float32
    run = pl.kernel(
        _edge_scores_body,
        out_type=(jax.ShapeDtypeStruct((E,), f32),
                  jax.ShapeDtypeStruct((E,), f32)),
        mesh=plsc.VectorSubcoreMesh(core_axis_name="c", subcore_axis_name="s"),
        compiler_params=pltpu.CompilerParams(needs_layout_passes=False,
                                             use_tc_tiling_on_sc=False),
        scratch_types=[
            pltpu.VMEM((EPW,), jnp.int32),
            pltpu.VMEM((EPW,), jnp.int32),
            pltpu.VMEM((EPW,), jnp.int32),
            pltpu.VMEM((2, 3, C, DP), jnp.int32),
            pltpu.VMEM((EPW,), f32),
            pltpu.VMEM((EPW,), f32),
            pltpu.SemaphoreType.DMA,
            pltpu.SemaphoreType.DMA,
        ],
    )
    return run(xs, xm, xr, a, b, c)


def kernel(x_srna, x_mrna, x_rbp, edge_label_index, edge_label_index_rbp):
    a = edge_label_index[0].astype(jnp.int32)
    b = edge_label_index[1].astype(jnp.int32)
    c = edge_label_index_rbp[0].astype(jnp.int32)
    return _edge_scores(_pack_table(x_srna), _pack_table(x_mrna),
                        _pack_table(x_rbp), a, b, c)


# speedup vs baseline: 1.3958x; 1.0481x over previous
import jax
import jax.numpy as jnp
from jax import lax
from jax.experimental import pallas as pl
from jax.experimental.pallas import tpu as pltpu
from jax.experimental.pallas import tpu_sc as plsc

N_NODES = 10000
D = 128
DP = D // 2
E = 320000

_INFO = plsc.get_sparse_core_info()
NC, NS, L = _INFO.num_cores, _INFO.num_subcores, _INFO.num_lanes
NW = NC * NS
EPW = E // NW
C = 200
NCHUNK = EPW // C


def _edge_scores_body(xs_hbm, xm_hbm, xr_hbm, a_hbm, b_hbm, c_hbm,
                      o1_hbm, o2_hbm,
                      ia, ib, ic, rows, o1v, o2v, sem0, sem1):
    wid = lax.axis_index("s") * NC + lax.axis_index("c")
    base_w = wid * EPW
    pltpu.sync_copy(a_hbm.at[pl.ds(base_w, EPW)], ia)
    pltpu.sync_copy(b_hbm.at[pl.ds(base_w, EPW)], ib)
    pltpu.sync_copy(c_hbm.at[pl.ds(base_w, EPW)], ic)
    sems = (sem0, sem1)
    tables = (xs_hbm, xm_hbm, xr_hbm)
    idxs = (ia, ib, ic)
    lane = lax.iota(jnp.int32, L)

    def copies(c, p):
        return [pltpu.make_async_copy(
                    tables[t].at[idxs[t].at[pl.ds(c * C, C)]],
                    rows.at[p, t], sems[p])
                for t in range(3)]

    def start(c, p):
        for cp in copies(c, p):
            cp.start()

    def compute(c, p):
        for cp in copies(c, p):
            cp.wait()
        rs, rm, rr = rows.at[p, 0], rows.at[p, 1], rows.at[p, 2]
        obase = c * C

        def group_body(g, carry):
            row = g * L + lane
            acc1 = jnp.zeros((L,), jnp.float32)
            acc2 = jnp.zeros((L,), jnp.float32)
            for d in range(DP):
                col = lane ^ d
                mp = plsc.load_gather(rm, [row, col])
                sp = plsc.load_gather(rs, [row, col])
                rp = plsc.load_gather(rr, [row, col])
                ma, mb = plsc.unpack(plsc.bitcast(mp, jnp.bfloat16),
                                     format=plsc.PackFormat.INTERLEAVED)
                sa, sb = plsc.unpack(plsc.bitcast(sp, jnp.bfloat16),
                                     format=plsc.PackFormat.INTERLEAVED)
                ra, rb = plsc.unpack(plsc.bitcast(rp, jnp.bfloat16),
                                     format=plsc.PackFormat.INTERLEAVED)
                acc1 = acc1 + sa * ma + sb * mb
                acc2 = acc2 + ra * ma + rb * mb
            o1v[pl.ds(g * L, L)] = acc1
            o2v[pl.ds(g * L, L)] = acc2
            return carry

        lax.fori_loop(0, C // L, group_body, 0)
        pltpu.sync_copy(o1v, o1_hbm.at[pl.ds(base_w + obase, C)])
        pltpu.sync_copy(o2v, o2_hbm.at[pl.ds(base_w + obase, C)])

    start(0, 0)

    def body2(k2, carry):
        c0 = 2 * k2
        start(c0 + 1, 1)
        compute(c0, 0)
        start(c0 + 2, 0)
        compute(c0 + 1, 1)
        return carry

    lax.fori_loop(0, (NCHUNK - 1) // 2, body2, 0)
    compute(NCHUNK - 1, 0)


def _pack_table(x):
    xb = x.astype(jnp.bfloat16).reshape(N_NODES, DP, 2)
    return jax.lax.bitcast_convert_type(xb, jnp.int32)


@jax.jit
def _edge_scores(xs, xm, xr, a, b, c):
    f32 = jnp.float32
    run = pl.kernel(
        _edge_scores_body,
        out_type=(jax.ShapeDtypeStruct((E,), f32),
                  jax.ShapeDtypeStruct((E,), f32)),
        mesh=plsc.VectorSubcoreMesh(core_axis_name="c", subcore_axis_name="s"),
        compiler_params=pltpu.CompilerParams(needs_layout_passes=False,
                                             use_tc_tiling_on_sc=False),
        scratch_types=[
            pltpu.VMEM((EPW,), jnp.int32),
            pltpu.VMEM((EPW,), jnp.int32),
            pltpu.VMEM((EPW,), jnp.int32),
            pltpu.VMEM((2, 3, C, DP), jnp.int32),
            pltpu.VMEM((C,), f32),
            pltpu.VMEM((C,), f32),
            pltpu.SemaphoreType.DMA,
            pltpu.SemaphoreType.DMA,
        ],
    )
    return run(xs, xm, xr, a, b, c)


def kernel(x_srna, x_mrna, x_rbp, edge_label_index, edge_label_index_rbp):
    a = edge_label_index[0].astype(jnp.int32)
    b = edge_label_index[1].astype(jnp.int32)
    c = edge_label_index_rbp[0].astype(jnp.int32)
    return _edge_scores(_pack_table(x_srna), _pack_table(x_mrna),
                        _pack_table(x_rbp), a, b, c)
